# Initial kernel scaffold; baseline (speedup 1.0000x reference)
import jax
import jax.numpy as jnp
from jax.experimental import pallas as pl


def kernel(indices, table):
    raise NotImplementedError("write your pallas kernel here")


# speedup vs baseline: 5.2143x; 5.2143x over previous
import functools

import jax
import jax.numpy as jnp
from jax import lax
from jax.experimental import pallas as pl
from jax.experimental.pallas import tpu as pltpu
from jax.experimental.pallas import tpu_sc as plsc

VOCAB = 100000
EMBED = 64
N_ROWS = 16384
N_COLS = 50
B_TOTAL = N_ROWS * N_COLS

NUM_CORES = 2
NUM_SUBCORES = 16
NW = NUM_CORES * NUM_SUBCORES
B_PER_W = B_TOTAL // NW
GROUP = 128
NGROUPS = B_PER_W // GROUP


def _make_gather():
    mesh = plsc.VectorSubcoreMesh(core_axis_name="c", subcore_axis_name="s")

    @functools.partial(
        pl.kernel,
        mesh=mesh,
        out_type=jax.ShapeDtypeStruct((B_TOTAL, EMBED), jnp.float32),
        scratch_types=[
            pltpu.VMEM((NGROUPS, GROUP), jnp.int32),
            pltpu.VMEM((GROUP, EMBED), jnp.float32),
            pltpu.SemaphoreType.DMA,
        ],
        compiler_params=pltpu.CompilerParams(use_tc_tiling_on_sc=False),
    )
    def gather_kernel(idx_hbm, table_hbm, out_hbm, idx_v, rows_v, sem):
        wid = lax.axis_index("s") * NUM_CORES + lax.axis_index("c")
        base = wid * B_PER_W
        pltpu.sync_copy(idx_hbm.at[pl.ds(wid * NGROUPS, NGROUPS)], idx_v)

        def body(j, carry):
            pltpu.async_copy(table_hbm.at[idx_v.at[j]], rows_v, sem).wait()
            pltpu.sync_copy(rows_v, out_hbm.at[pl.ds(base + j * GROUP, GROUP)])
            return carry

        lax.fori_loop(0, NGROUPS, body, 0)

    return gather_kernel


_gather = _make_gather()


def kernel(indices, table):
    idx2d = indices.reshape(NW * NGROUPS, GROUP).astype(jnp.int32)
    out = _gather(idx2d, table)
    return out.reshape(N_ROWS, N_COLS, EMBED)


# speedup vs baseline: 6.2062x; 1.1902x over previous
import functools

import jax
import jax.numpy as jnp
from jax import lax
from jax.experimental import pallas as pl
from jax.experimental.pallas import tpu as pltpu
from jax.experimental.pallas import tpu_sc as plsc

VOCAB = 100000
EMBED = 64
N_ROWS = 16384
N_COLS = 50
B_TOTAL = N_ROWS * N_COLS

NUM_CORES = 2
NUM_SUBCORES = 16
NW = NUM_CORES * NUM_SUBCORES
B_PER_W = B_TOTAL // NW
GROUP = 128
NGROUPS = B_PER_W // GROUP
K = 4
NPAIRS = NGROUPS // (2 * K)


def _make_gather():
    mesh = plsc.VectorSubcoreMesh(core_axis_name="c", subcore_axis_name="s")

    @functools.partial(
        pl.kernel,
        mesh=mesh,
        out_type=jax.ShapeDtypeStruct((B_TOTAL, EMBED), jnp.float32),
        scratch_types=[
            pltpu.VMEM((NGROUPS, GROUP), jnp.int32),
            pltpu.VMEM((K, GROUP, EMBED), jnp.float32),
            pltpu.VMEM((K, GROUP, EMBED), jnp.float32),
            pltpu.SemaphoreType.DMA,
            pltpu.SemaphoreType.DMA,
            pltpu.SemaphoreType.DMA,
            pltpu.SemaphoreType.DMA,
        ],
        compiler_params=pltpu.CompilerParams(use_tc_tiling_on_sc=False),
    )
    def gather_kernel(idx_hbm, table_hbm, out_hbm, idx_v, rows_a, rows_b,
                      gsem_a, gsem_b, ssem_a, ssem_b):
        wid = lax.axis_index("s") * NUM_CORES + lax.axis_index("c")
        base = wid * B_PER_W
        pltpu.sync_copy(idx_hbm.at[pl.ds(wid * NGROUPS, NGROUPS)], idx_v)

        def fg(rows, gsem, t):
            for b in range(K):
                pltpu.async_copy(
                    table_hbm.at[idx_v.at[t * K + b]], rows.at[b], gsem)

        def dg(rows, gsem):
            for b in range(K):
                pltpu.make_async_copy(
                    table_hbm.at[pl.ds(0, GROUP)], rows.at[b], gsem).wait()

        def fs(rows, ssem, t):
            for b in range(K):
                j = t * K + b
                pltpu.async_copy(
                    rows.at[b], out_hbm.at[pl.ds(base + j * GROUP, GROUP)],
                    ssem)

        def ds(rows, ssem):
            for b in range(K):
                pltpu.make_async_copy(
                    rows.at[b], out_hbm.at[pl.ds(0, GROUP)], ssem).wait()

        fg(rows_a, gsem_a, 0)
        dg(rows_a, gsem_a)
        fg(rows_b, gsem_b, 1)
        fs(rows_a, ssem_a, 0)
        dg(rows_b, gsem_b)
        ds(rows_a, ssem_a)
        fg(rows_a, gsem_a, 2)
        fs(rows_b, ssem_b, 1)

        def body(p, carry):
            dg(rows_a, gsem_a)
            ds(rows_b, ssem_b)
            fg(rows_b, gsem_b, 2 * p + 1)
            fs(rows_a, ssem_a, 2 * p)
            dg(rows_b, gsem_b)
            ds(rows_a, ssem_a)
            fg(rows_a, gsem_a, 2 * p + 2)
            fs(rows_b, ssem_b, 2 * p + 1)
            return carry

        lax.fori_loop(1, NPAIRS - 1, body, 0)

        t1 = 2 * NPAIRS - 1
        dg(rows_a, gsem_a)
        ds(rows_b, ssem_b)
        fg(rows_b, gsem_b, t1)
        fs(rows_a, ssem_a, t1 - 1)
        dg(rows_b, gsem_b)
        ds(rows_a, ssem_a)
        fs(rows_b, ssem_b, t1)
        ds(rows_b, ssem_b)

    return gather_kernel


_gather = _make_gather()


def kernel(indices, table):
    idx2d = indices.reshape(NW * NGROUPS, GROUP).astype(jnp.int32)
    out = _gather(idx2d, table)
    return out.reshape(N_ROWS, N_COLS, EMBED)
